# Initial kernel scaffold; baseline (speedup 1.0000x reference)
import jax
import jax.numpy as jnp
from jax.experimental import pallas as pl


def kernel(x, edge_attr, params, edge_index):
    raise NotImplementedError("write your pallas kernel here")


# speedup vs baseline: 3.1034x; 3.1034x over previous
import functools

import jax
import jax.numpy as jnp
from jax import lax
from jax.experimental import pallas as pl
from jax.experimental.pallas import tpu as pltpu
from jax.experimental.pallas import tpu_sc as plsc

NN = 10000
NE = 320000
H = 128
CHUNK = 128
NCH = NE // CHUNK
NW = 32
ITERS = -(-NCH // NW)
ROWS_PER_TILE = NN // 16

_f32 = jnp.float32


@functools.cache
def _sc_kernels():
    mesh = plsc.VectorSubcoreMesh(
        core_axis_name="c", subcore_axis_name="s", num_cores=2,
        num_subcores=16)

    @functools.partial(
        pl.kernel,
        out_type=(jax.ShapeDtypeStruct((NE, H), _f32),
                  jax.ShapeDtypeStruct((NE, H), _f32)),
        mesh=mesh,
        scratch_types=[
            pltpu.VMEM((CHUNK,), jnp.int32),
            pltpu.VMEM((CHUNK, H), _f32),
            pltpu.VMEM((CHUNK,), jnp.int32),
            pltpu.VMEM((CHUNK, H), _f32),
            pltpu.SemaphoreType.DMA,
            pltpu.SemaphoreType.DMA,
        ],
    )
    def sc_gather2(hs_hbm, hr_hbm, snd_hbm, rcv_hbm, gs_hbm, gr_hbm,
                   idx_s, rows_s, idx_r, rows_r, sem_s, sem_r):
        wid = lax.axis_index("s") * 2 + lax.axis_index("c")

        def body(i, carry):
            k = wid + i * NW

            @pl.when(k < NCH)
            def _():
                base = pl.multiple_of(k * CHUNK, CHUNK)
                pltpu.sync_copy(snd_hbm.at[pl.ds(base, CHUNK)], idx_s)
                pltpu.sync_copy(rcv_hbm.at[pl.ds(base, CHUNK)], idx_r)
                a = pltpu.async_copy(hs_hbm.at[idx_s], rows_s, sem_s)
                b = pltpu.async_copy(hr_hbm.at[idx_r], rows_r, sem_r)
                a.wait()
                b.wait()
                pltpu.sync_copy(rows_s, gs_hbm.at[pl.ds(base, CHUNK)])
                pltpu.sync_copy(rows_r, gr_hbm.at[pl.ds(base, CHUNK)])

            return carry

        lax.fori_loop(0, ITERS, body, 0)

    @functools.partial(
        pl.kernel,
        out_type=jax.ShapeDtypeStruct((2, NN, H), _f32),
        mesh=mesh,
        scratch_types=[
            pltpu.VMEM((CHUNK,), jnp.int32),
            pltpu.VMEM((CHUNK, H), _f32),
            pltpu.VMEM_SHARED((NN, H), _f32),
        ],
    )
    def sc_scatter(enew_hbm, rcv_hbm, zeros_hbm, agg_hbm, idx_v, rows_v,
                   shared):
        c = lax.axis_index("c")
        s = lax.axis_index("s")
        wid = s * 2 + c
        base_t = pl.multiple_of(s * 624, 8)

        @pl.when(s < 15)
        def _():
            pltpu.sync_copy(zeros_hbm.at[pl.ds(base_t, 624)],
                            shared.at[pl.ds(base_t, 624)])

        @pl.when(s == 15)
        def _():
            pltpu.sync_copy(zeros_hbm.at[pl.ds(9360, 640)],
                            shared.at[pl.ds(9360, 640)])

        plsc.subcore_barrier()

        def body(i, carry):
            k = wid + i * NW

            @pl.when(k < NCH)
            def _():
                base = pl.multiple_of(k * CHUNK, CHUNK)
                pltpu.sync_copy(rcv_hbm.at[pl.ds(base, CHUNK)], idx_v)
                pltpu.sync_copy(enew_hbm.at[pl.ds(base, CHUNK)], rows_v)
                pltpu.sync_copy(rows_v, shared.at[idx_v], add=True)

            return carry

        lax.fori_loop(0, ITERS, body, 0)
        plsc.subcore_barrier()

        @pl.when(s < 15)
        def _():
            pltpu.sync_copy(shared.at[pl.ds(base_t, 624)],
                            agg_hbm.at[c].at[pl.ds(base_t, 624)])

        @pl.when(s == 15)
        def _():
            pltpu.sync_copy(shared.at[pl.ds(9360, 640)],
                            agg_hbm.at[c].at[pl.ds(9360, 640)])

    return sc_gather2, sc_scatter


def _dot(a, b):
    return jnp.dot(a, b, preferred_element_type=_f32)


def _mlp_tail(h1_pre, W2, b2, W3, b3, W4, b4):
    h = jnp.maximum(h1_pre, 0.0)
    h = jnp.maximum(_dot(h, W2) + b2, 0.0)
    h = jnp.maximum(_dot(h, W3) + b3, 0.0)
    return _dot(h, W4) + b4


def _ln(h, g, beta):
    mu = jnp.mean(h, axis=-1, keepdims=True)
    var = jnp.mean((h - mu) * (h - mu), axis=-1, keepdims=True)
    return (h - mu) * lax.rsqrt(var + 1e-5) * g + beta


def _enc_node_body(x, W1, b1, W2, b2, W3, b3, W4, b4, g, beta, Ws, Wr,
                   node_o, hs_o, hr_o):
    pre = _dot(x[...], W1[...]) + b1[...]
    h = _mlp_tail(pre, W2[...], b2[...], W3[...], b3[...], W4[...], b4[...])
    n = _ln(h, g[...], beta[...])
    node_o[...] = n
    hs_o[...] = _dot(n, Ws[...])
    hr_o[...] = _dot(n, Wr[...])


def _enc_edge_body(ea, W1, b1, W2, b2, W3, b3, W4, b4, g, beta, edge_o):
    pre = _dot(ea[...], W1[...]) + b1[...]
    h = _mlp_tail(pre, W2[...], b2[...], W3[...], b3[...], W4[...], b4[...])
    edge_o[...] = _ln(h, g[...], beta[...])


def _edge_block_body(with_res, e, gs, gr, We, b1, W2, b2, W3, b3, W4, b4,
                     g, beta, *outs):
    pre = _dot(e[...], We[...]) + gs[...] + gr[...] + b1[...]
    h = _mlp_tail(pre, W2[...], b2[...], W3[...], b3[...], W4[...], b4[...])
    en = _ln(h, g[...], beta[...])
    outs[0][...] = en
    if with_res:
        outs[1][...] = e[...] + en


def _node_block_body(with_next, n, aa, ab, Wn, Wa, b1, W2, b2, W3, b3, W4, b4,
                     g, beta, *rest):
    agg = aa[...] + ab[...]
    pre = _dot(n[...], Wn[...]) + _dot(agg, Wa[...]) + b1[...]
    h = _mlp_tail(pre, W2[...], b2[...], W3[...], b3[...], W4[...], b4[...])
    nn = n[...] + _ln(h, g[...], beta[...])
    if with_next:
        Ws, Wr, node_o, hs_o, hr_o = rest
        node_o[...] = nn
        hs_o[...] = _dot(nn, Ws[...])
        hr_o[...] = _dot(nn, Wr[...])
    else:
        rest[0][...] = nn


def _decoder_body(n, W1, b1, W2, b2, W3, b3, W4, b4, out_o):
    pre = _dot(n[...], W1[...]) + b1[...]
    out_o[...] = _mlp_tail(pre, W2[...], b2[...], W3[...], b3[...], W4[...],
                           b4[...])


def _call_rowwise(body, row_ins, w_ins, out_dims, tile):
    n = row_ins[0].shape[0]
    grid = n // tile
    in_specs = (
        [pl.BlockSpec((tile, a.shape[1]), lambda i: (i, 0)) for a in row_ins]
        + [pl.BlockSpec(w.shape, lambda i, nd=w.ndim: (0,) * nd)
           for w in w_ins])
    out_specs = [pl.BlockSpec((tile, d), lambda i: (i, 0)) for d in out_dims]
    out_shape = [jax.ShapeDtypeStruct((n, d), _f32) for d in out_dims]
    res = pl.pallas_call(
        body,
        grid=(grid,),
        in_specs=in_specs,
        out_specs=out_specs,
        out_shape=out_shape,
        compiler_params=pltpu.CompilerParams(
            dimension_semantics=("arbitrary",)),
    )(*row_ins, *w_ins)
    return res


def _mlp_ws(p):
    out = []
    for i in (1, 2, 3):
        out.append(p["W"][i])
        out.append(p["b"][i].reshape(1, -1))
    if "g" in p:
        out.append(p["g"].reshape(1, -1))
        out.append(p["beta"].reshape(1, -1))
    return out


def kernel(x, edge_attr, params, edge_index):
    senders = edge_index[0]
    receivers = edge_index[1]
    blocks = params["blocks"]
    zeros = jnp.zeros((NN, H), _f32)
    _sc_gather2, _sc_scatter = _sc_kernels()

    def split_eb_w1(blk):
        W1 = blk["eb"]["W"][0]
        return W1[:H], W1[H:2 * H], W1[2 * H:]

    Ws1, Wr1, _ = split_eb_w1(blocks[0])

    p = params["nb_enc"]
    node, hs, hr = _call_rowwise(
        _enc_node_body, [x],
        [p["W"][0], p["b"][0].reshape(1, -1)] + _mlp_ws(p) + [Ws1, Wr1],
        [H, H, H], 2000)

    p = params["eb_enc"]
    edge = _call_rowwise(
        _enc_edge_body, [edge_attr],
        [p["W"][0], p["b"][0].reshape(1, -1)] + _mlp_ws(p),
        [H], 2000)[0]

    for i, blk in enumerate(blocks):
        last = i == len(blocks) - 1
        gs, gr = _sc_gather2(hs, hr, senders, receivers)

        _, _, We = split_eb_w1(blk)
        pe = blk["eb"]
        eouts = _call_rowwise(
            functools.partial(_edge_block_body, not last),
            [edge, gs, gr],
            [We, pe["b"][0].reshape(1, -1)] + _mlp_ws(pe),
            [H, H] if not last else [H], 2000)
        enew = eouts[0]

        agg = _sc_scatter(enew, receivers, zeros)

        pn = blk["nb"]
        Wn = pn["W"][0][:H]
        Wa = pn["W"][0][H:]
        nb_w = [Wn, Wa, pn["b"][0].reshape(1, -1)] + _mlp_ws(pn)
        if not last:
            Wsn, Wrn, _ = split_eb_w1(blocks[i + 1])
            node, hs, hr = _call_rowwise(
                functools.partial(_node_block_body, True),
                [node, agg[0], agg[1]], nb_w + [Wsn, Wrn], [H, H, H], 2000)
            edge = eouts[1]
        else:
            node = _call_rowwise(
                functools.partial(_node_block_body, False),
                [node, agg[0], agg[1]], nb_w, [H], 2000)[0]

    p = params["decoder"]
    W4 = jnp.zeros((H, H), _f32).at[:, :3].set(p["W"][3])
    b4 = jnp.zeros((1, H), _f32).at[0, :3].set(p["b"][3])
    dec_w = [p["W"][0], p["b"][0].reshape(1, -1)]
    for i in (1, 2):
        dec_w += [p["W"][i], p["b"][i].reshape(1, -1)]
    dec_w += [W4, b4]
    out = _call_rowwise(_decoder_body, [node], dec_w, [H], 2000)[0]
    return out[:, :3]


# speedup vs baseline: 3.5735x; 1.1515x over previous
import functools

import jax
import jax.numpy as jnp
from jax import lax
from jax.experimental import pallas as pl
from jax.experimental.pallas import tpu as pltpu
from jax.experimental.pallas import tpu_sc as plsc

NN = 10000
NE = 320000
H = 128
CHUNK = 128
NCH = NE // CHUNK
NW = 32
ITERS = -(-NCH // NW)
ROWS_PER_TILE = NN // 16

_f32 = jnp.float32


@functools.cache
def _sc_kernels():
    mesh = plsc.VectorSubcoreMesh(
        core_axis_name="c", subcore_axis_name="s", num_cores=2,
        num_subcores=16)

    @functools.partial(
        pl.kernel,
        out_type=(jax.ShapeDtypeStruct((NE, H), _f32),
                  jax.ShapeDtypeStruct((NE, H), _f32)),
        mesh=mesh,
        scratch_types=[
            pltpu.VMEM((2, CHUNK), jnp.int32),
            pltpu.VMEM((2, CHUNK, H), _f32),
            pltpu.VMEM((2, CHUNK), jnp.int32),
            pltpu.VMEM((2, CHUNK, H), _f32),
            pltpu.SemaphoreType.DMA,
            pltpu.SemaphoreType.DMA,
            pltpu.SemaphoreType.DMA,
        ],
    )
    def sc_gather2(hs_hbm, hr_hbm, snd_hbm, rcv_hbm, gs_hbm, gr_hbm,
                   idx_s, rows_s, idx_r, rows_r, sem_i, sem_g, sem_w):
        wid = lax.axis_index("s") * 2 + lax.axis_index("c")

        def body(i, carry):
            ds = []
            for b in range(2):
                k = wid + (2 * i + b) * NW
                base = pl.multiple_of(k * CHUNK, CHUNK)
                ds.append(base)
                pltpu.async_copy(snd_hbm.at[pl.ds(base, CHUNK)],
                                 idx_s.at[b], sem_i)
                pltpu.async_copy(rcv_hbm.at[pl.ds(base, CHUNK)],
                                 idx_r.at[b], sem_i)
            for b in range(2):
                pltpu.make_async_copy(snd_hbm.at[pl.ds(ds[b], CHUNK)],
                                      idx_s.at[b], sem_i).wait()
                pltpu.make_async_copy(rcv_hbm.at[pl.ds(ds[b], CHUNK)],
                                      idx_r.at[b], sem_i).wait()
            gs_d = []
            for b in range(2):
                gs_d.append(pltpu.async_copy(hs_hbm.at[idx_s.at[b]],
                                             rows_s.at[b], sem_g))
                gs_d.append(pltpu.async_copy(hr_hbm.at[idx_r.at[b]],
                                             rows_r.at[b], sem_g))
            for d in gs_d:
                d.wait()
            w_d = []
            for b in range(2):
                w_d.append(pltpu.async_copy(
                    rows_s.at[b], gs_hbm.at[pl.ds(ds[b], CHUNK)], sem_w))
                w_d.append(pltpu.async_copy(
                    rows_r.at[b], gr_hbm.at[pl.ds(ds[b], CHUNK)], sem_w))
            for d in w_d:
                d.wait()
            return carry

        lax.fori_loop(0, (NCH // NW) // 2, body, 0)

        @pl.when(wid < NCH - NW * (NCH // NW))
        def _():
            k = NW * (NCH // NW) + wid
            base = pl.multiple_of(k * CHUNK, CHUNK)
            pltpu.sync_copy(snd_hbm.at[pl.ds(base, CHUNK)], idx_s.at[0])
            pltpu.sync_copy(rcv_hbm.at[pl.ds(base, CHUNK)], idx_r.at[0])
            a = pltpu.async_copy(hs_hbm.at[idx_s.at[0]], rows_s.at[0], sem_g)
            b = pltpu.async_copy(hr_hbm.at[idx_r.at[0]], rows_r.at[0], sem_g)
            a.wait()
            b.wait()
            pltpu.sync_copy(rows_s.at[0], gs_hbm.at[pl.ds(base, CHUNK)])
            pltpu.sync_copy(rows_r.at[0], gr_hbm.at[pl.ds(base, CHUNK)])

    @functools.partial(
        pl.kernel,
        out_type=jax.ShapeDtypeStruct((2, NN, H), _f32),
        mesh=mesh,
        scratch_types=[
            pltpu.VMEM((2, CHUNK), jnp.int32),
            pltpu.VMEM((2, CHUNK, H), _f32),
            pltpu.VMEM_SHARED((NN, H), _f32),
            pltpu.SemaphoreType.DMA,
            pltpu.SemaphoreType.DMA,
        ],
    )
    def sc_scatter(enew_hbm, rcv_hbm, zeros_hbm, agg_hbm, idx_v, rows_v,
                   shared, sem_l, sem_a):
        c = lax.axis_index("c")
        s = lax.axis_index("s")
        wid = s * 2 + c
        base_t = pl.multiple_of(s * 624, 8)

        @pl.when(s < 15)
        def _():
            pltpu.sync_copy(zeros_hbm.at[pl.ds(base_t, 624)],
                            shared.at[pl.ds(base_t, 624)])

        @pl.when(s == 15)
        def _():
            pltpu.sync_copy(zeros_hbm.at[pl.ds(9360, 640)],
                            shared.at[pl.ds(9360, 640)])

        plsc.subcore_barrier()

        def body(i, carry):
            l_d = []
            for b in range(2):
                k = wid + (2 * i + b) * NW
                base = pl.multiple_of(k * CHUNK, CHUNK)
                l_d.append(pltpu.async_copy(rcv_hbm.at[pl.ds(base, CHUNK)],
                                            idx_v.at[b], sem_l))
                l_d.append(pltpu.async_copy(enew_hbm.at[pl.ds(base, CHUNK)],
                                            rows_v.at[b], sem_l))
            for d in l_d:
                d.wait()
            a_d = []
            for b in range(2):
                a_d.append(pltpu.async_copy(rows_v.at[b],
                                            shared.at[idx_v.at[b]], sem_a,
                                            add=True))
            for d in a_d:
                d.wait()
            return carry

        lax.fori_loop(0, (NCH // NW) // 2, body, 0)

        @pl.when(wid < NCH - NW * (NCH // NW))
        def _():
            k = NW * (NCH // NW) + wid
            base = pl.multiple_of(k * CHUNK, CHUNK)
            pltpu.sync_copy(rcv_hbm.at[pl.ds(base, CHUNK)], idx_v.at[0])
            pltpu.sync_copy(enew_hbm.at[pl.ds(base, CHUNK)], rows_v.at[0])
            pltpu.sync_copy(rows_v.at[0], shared.at[idx_v.at[0]], add=True)

        plsc.subcore_barrier()

        @pl.when(s < 15)
        def _():
            pltpu.sync_copy(shared.at[pl.ds(base_t, 624)],
                            agg_hbm.at[c].at[pl.ds(base_t, 624)])

        @pl.when(s == 15)
        def _():
            pltpu.sync_copy(shared.at[pl.ds(9360, 640)],
                            agg_hbm.at[c].at[pl.ds(9360, 640)])

    return sc_gather2, sc_scatter


def _dot(a, b):
    return jnp.dot(a, b, preferred_element_type=_f32)


def _mlp_tail(h1_pre, W2, b2, W3, b3, W4, b4):
    h = jnp.maximum(h1_pre, 0.0)
    h = jnp.maximum(_dot(h, W2) + b2, 0.0)
    h = jnp.maximum(_dot(h, W3) + b3, 0.0)
    return _dot(h, W4) + b4


def _ln(h, g, beta):
    mu = jnp.mean(h, axis=-1, keepdims=True)
    var = jnp.mean((h - mu) * (h - mu), axis=-1, keepdims=True)
    return (h - mu) * lax.rsqrt(var + 1e-5) * g + beta


def _enc_node_body(x, W1, b1, W2, b2, W3, b3, W4, b4, g, beta, Ws, Wr,
                   node_o, hs_o, hr_o):
    pre = _dot(x[...], W1[...]) + b1[...]
    h = _mlp_tail(pre, W2[...], b2[...], W3[...], b3[...], W4[...], b4[...])
    n = _ln(h, g[...], beta[...])
    node_o[...] = n
    hs_o[...] = _dot(n, Ws[...])
    hr_o[...] = _dot(n, Wr[...])


def _enc_edge_body(ea, W1, b1, W2, b2, W3, b3, W4, b4, g, beta, edge_o):
    pre = _dot(ea[...], W1[...]) + b1[...]
    h = _mlp_tail(pre, W2[...], b2[...], W3[...], b3[...], W4[...], b4[...])
    edge_o[...] = _ln(h, g[...], beta[...])


def _edge_block_body(with_res, e, gs, gr, We, b1, W2, b2, W3, b3, W4, b4,
                     g, beta, *outs):
    pre = _dot(e[...], We[...]) + gs[...] + gr[...] + b1[...]
    h = _mlp_tail(pre, W2[...], b2[...], W3[...], b3[...], W4[...], b4[...])
    en = _ln(h, g[...], beta[...])
    outs[0][...] = en
    if with_res:
        outs[1][...] = e[...] + en


def _node_block_body(with_next, n, aa, ab, Wn, Wa, b1, W2, b2, W3, b3, W4, b4,
                     g, beta, *rest):
    agg = aa[...] + ab[...]
    pre = _dot(n[...], Wn[...]) + _dot(agg, Wa[...]) + b1[...]
    h = _mlp_tail(pre, W2[...], b2[...], W3[...], b3[...], W4[...], b4[...])
    nn = n[...] + _ln(h, g[...], beta[...])
    if with_next:
        Ws, Wr, node_o, hs_o, hr_o = rest
        node_o[...] = nn
        hs_o[...] = _dot(nn, Ws[...])
        hr_o[...] = _dot(nn, Wr[...])
    else:
        rest[0][...] = nn


def _decoder_body(n, W1, b1, W2, b2, W3, b3, W4, b4, out_o):
    pre = _dot(n[...], W1[...]) + b1[...]
    out_o[...] = _mlp_tail(pre, W2[...], b2[...], W3[...], b3[...], W4[...],
                           b4[...])


def _call_rowwise(body, row_ins, w_ins, out_dims, tile):
    n = row_ins[0].shape[0]
    grid = n // tile
    in_specs = (
        [pl.BlockSpec((tile, a.shape[1]), lambda i: (i, 0)) for a in row_ins]
        + [pl.BlockSpec(w.shape, lambda i, nd=w.ndim: (0,) * nd)
           for w in w_ins])
    out_specs = [pl.BlockSpec((tile, d), lambda i: (i, 0)) for d in out_dims]
    out_shape = [jax.ShapeDtypeStruct((n, d), _f32) for d in out_dims]
    res = pl.pallas_call(
        body,
        grid=(grid,),
        in_specs=in_specs,
        out_specs=out_specs,
        out_shape=out_shape,
        compiler_params=pltpu.CompilerParams(
            dimension_semantics=("arbitrary",)),
    )(*row_ins, *w_ins)
    return res


def _mlp_ws(p):
    out = []
    for i in (1, 2, 3):
        out.append(p["W"][i])
        out.append(p["b"][i].reshape(1, -1))
    if "g" in p:
        out.append(p["g"].reshape(1, -1))
        out.append(p["beta"].reshape(1, -1))
    return out


def kernel(x, edge_attr, params, edge_index):
    senders = edge_index[0]
    receivers = edge_index[1]
    blocks = params["blocks"]
    zeros = jnp.zeros((NN, H), _f32)
    _sc_gather2, _sc_scatter = _sc_kernels()

    def split_eb_w1(blk):
        W1 = blk["eb"]["W"][0]
        return W1[:H], W1[H:2 * H], W1[2 * H:]

    Ws1, Wr1, _ = split_eb_w1(blocks[0])

    p = params["nb_enc"]
    node, hs, hr = _call_rowwise(
        _enc_node_body, [x],
        [p["W"][0], p["b"][0].reshape(1, -1)] + _mlp_ws(p) + [Ws1, Wr1],
        [H, H, H], 2000)

    p = params["eb_enc"]
    edge = _call_rowwise(
        _enc_edge_body, [edge_attr],
        [p["W"][0], p["b"][0].reshape(1, -1)] + _mlp_ws(p),
        [H], 2000)[0]

    for i, blk in enumerate(blocks):
        last = i == len(blocks) - 1
        gs, gr = _sc_gather2(hs, hr, senders, receivers)

        _, _, We = split_eb_w1(blk)
        pe = blk["eb"]
        eouts = _call_rowwise(
            functools.partial(_edge_block_body, not last),
            [edge, gs, gr],
            [We, pe["b"][0].reshape(1, -1)] + _mlp_ws(pe),
            [H, H] if not last else [H], 2000)
        enew = eouts[0]

        agg = _sc_scatter(enew, receivers, zeros)

        pn = blk["nb"]
        Wn = pn["W"][0][:H]
        Wa = pn["W"][0][H:]
        nb_w = [Wn, Wa, pn["b"][0].reshape(1, -1)] + _mlp_ws(pn)
        if not last:
            Wsn, Wrn, _ = split_eb_w1(blocks[i + 1])
            node, hs, hr = _call_rowwise(
                functools.partial(_node_block_body, True),
                [node, agg[0], agg[1]], nb_w + [Wsn, Wrn], [H, H, H], 2000)
            edge = eouts[1]
        else:
            node = _call_rowwise(
                functools.partial(_node_block_body, False),
                [node, agg[0], agg[1]], nb_w, [H], 2000)[0]

    p = params["decoder"]
    W4 = jnp.zeros((H, H), _f32).at[:, :3].set(p["W"][3])
    b4 = jnp.zeros((1, H), _f32).at[0, :3].set(p["b"][3])
    dec_w = [p["W"][0], p["b"][0].reshape(1, -1)]
    for i in (1, 2):
        dec_w += [p["W"][i], p["b"][i].reshape(1, -1)]
    dec_w += [W4, b4]
    out = _call_rowwise(_decoder_body, [node], dec_w, [H], 2000)[0]
    return out[:, :3]
